# baseline (speedup 1.0000x reference)
import functools

import jax
import jax.numpy as jnp
from jax import lax
from jax.experimental import pallas as pl
from jax.experimental.pallas import tpu as pltpu
from jax.experimental.pallas import tpu_sc as plsc

NC = 2
NS = 16
NW = NC * NS
L = 16
B = 16384
K = 32
BPW = B // NW
SUB = 128
NSUB = BPW // SUB

_mesh = plsc.VectorSubcoreMesh(core_axis_name="c", subcore_axis_name="s")


@functools.partial(
    pl.kernel,
    mesh=_mesh,
    compiler_params=pltpu.CompilerParams(use_tc_tiling_on_sc=False),
    out_type=[
        jax.ShapeDtypeStruct((B,), jnp.float32),
        jax.ShapeDtypeStruct((B, K), jnp.float32),
        jax.ShapeDtypeStruct((B, K), jnp.float32),
    ],
    scratch_types=[
        pltpu.VMEM((NSUB, SUB), jnp.int32),
        pltpu.VMEM((NSUB, SUB), jnp.int32),
        pltpu.VMEM((BPW, K), jnp.float32),
        pltpu.VMEM((BPW, K), jnp.float32),
        pltpu.VMEM((BPW,), jnp.float32),
        pltpu.SemaphoreType.DMA,
        pltpu.SemaphoreType.DMA,
    ],
)
def _mf_sc(uidx_hbm, vidx_hbm, w_hbm, h_hbm,
           out_hbm, ue_hbm, ve_hbm,
           uidx_v, vidx_v, u_v, v_v, o_v, gsem, wsem):
    wid = lax.axis_index("s") * NC + lax.axis_index("c")
    base = wid * BPW

    pltpu.sync_copy(uidx_hbm.at[pl.ds(wid * NSUB, NSUB)], uidx_v)
    pltpu.sync_copy(vidx_hbm.at[pl.ds(wid * NSUB, NSUB)], vidx_v)

    copies = []
    for j in range(NSUB):
        copies.append(pltpu.async_copy(
            w_hbm.at[uidx_v.at[j]], u_v.at[pl.ds(j * SUB, SUB)], gsem))
        copies.append(pltpu.async_copy(
            h_hbm.at[vidx_v.at[j]], v_v.at[pl.ds(j * SUB, SUB)], gsem))
    for c in copies:
        c.wait()

    ue_copy = pltpu.async_copy(u_v, ue_hbm.at[pl.ds(base, BPW)], wsem)
    ve_copy = pltpu.async_copy(v_v, ve_hbm.at[pl.ds(base, BPW)], wsem)

    lanes = lax.iota(jnp.int32, L)
    perm_idx = [lanes ^ sh for sh in (8, 4, 2, 1)]
    lane_masks = [lanes == j for j in range(L)]

    def _perm(x, idx):
        return lax.gather(
            x, idx[:, None],
            lax.GatherDimensionNumbers(
                offset_dims=(), collapsed_slice_dims=(0,),
                start_index_map=(0,)),
            (1,), mode=lax.GatherScatterMode.PROMISE_IN_BOUNDS)

    def group_body(g, carry):
        acc = jnp.zeros((L,), jnp.float32)
        for j in range(L):
            r = g * L + j
            u0 = u_v[r, pl.ds(0, L)]
            u1 = u_v[r, pl.ds(L, L)]
            v0 = v_v[r, pl.ds(0, L)]
            v1 = v_v[r, pl.ds(L, L)]
            p = u0 * v0 + u1 * v1
            for idx in perm_idx:
                p = p + _perm(p, idx)
            acc = jnp.where(lane_masks[j], p, acc)
        o_v[pl.ds(g * L, L)] = 1.0 / (1.0 + jnp.exp(-acc))
        return carry

    lax.fori_loop(0, BPW // L, group_body, 0)

    pltpu.sync_copy(o_v, out_hbm.at[pl.ds(base, BPW)])
    ue_copy.wait()
    ve_copy.wait()


def kernel(x, W, H):
    uidx = x[:, 0].astype(jnp.int32).reshape(NW * NSUB, SUB)
    vidx = x[:, 1].astype(jnp.int32).reshape(NW * NSUB, SUB)
    out, ue, ve = _mf_sc(uidx, vidx, W, H)
    return out, ue, ve
